# baseline (speedup 1.0000x reference)
import functools

import jax
import jax.numpy as jnp
from jax import lax
from jax.experimental import pallas as pl
from jax.experimental.pallas import tpu as pltpu
from jax.experimental.pallas import tpu_sc as plsc

N = 10000
D = 128
E = 320000

NC = 2
NS = 16
NW = NC * NS
NPAD = 10240
CHUNK = 80
NSTEP = E // CHUNK // NW
RPS = NPAD // NS
PRPS = N // NS
F32 = jnp.float32

_mesh = plsc.VectorSubcoreMesh(core_axis_name="c", subcore_axis_name="s")


@functools.partial(
    pl.kernel,
    out_type=(
        jax.ShapeDtypeStruct((NPAD,), F32),
        jax.ShapeDtypeStruct((NPAD,), F32),
    ),
    mesh=_mesh,
    scratch_types=[
        pltpu.VMEM((NSTEP, CHUNK), jnp.int32),
        pltpu.VMEM((CHUNK,), F32),
        pltpu.VMEM((RPS,), F32),
        pltpu.VMEM_SHARED((NPAD,), F32),
    ],
)
def _sc_degree(dst_ref, out0, out1, idx_d, ones, zbuf, degacc):
    c = lax.axis_index("c")
    s = lax.axis_index("s")
    w = c * NS + s
    pltpu.sync_copy(dst_ref.at[w], idx_d)

    one16 = jnp.ones((16,), F32)
    zero16 = jnp.zeros((16,), F32)
    for l in range(CHUNK // 16):
        ones[pl.ds(l * 16, 16)] = one16

    def zfill(k, carry):
        zbuf[pl.ds(pl.multiple_of(k * 16, 16), 16)] = zero16
        return carry

    lax.fori_loop(0, RPS // 16, zfill, 0)
    pltpu.sync_copy(zbuf, degacc.at[pl.ds(s * RPS, RPS)])
    plsc.subcore_barrier()

    def body(i, carry):
        pltpu.sync_copy(ones, degacc.at[idx_d.at[i]], add=True)
        return carry

    lax.fori_loop(0, NSTEP, body, 0)
    plsc.subcore_barrier()

    pltpu.sync_copy(degacc.at[pl.ds(s * RPS, RPS)], zbuf)

    @pl.when(c == 0)
    def _():
        pltpu.sync_copy(zbuf, out0.at[pl.ds(s * RPS, RPS)])

    @pl.when(c == 1)
    def _():
        pltpu.sync_copy(zbuf, out1.at[pl.ds(s * RPS, RPS)])


@functools.partial(
    pl.kernel,
    out_type=(
        jax.ShapeDtypeStruct((NPAD, D), F32),
        jax.ShapeDtypeStruct((NPAD, D), F32),
    ),
    mesh=_mesh,
    scratch_types=[
        pltpu.VMEM((NSTEP * CHUNK,), jnp.int32),
        pltpu.VMEM((NSTEP, CHUNK), jnp.int32),
        pltpu.VMEM((CHUNK, D), F32),
        pltpu.VMEM((CHUNK, D), F32),
        pltpu.VMEM_SHARED((NPAD, D), F32),
        pltpu.SemaphoreType.DMA,
        pltpu.SemaphoreType.DMA,
        pltpu.SemaphoreType.DMA,
    ],
)
def _sc_prop(src_ref, dst_ref, g_ref, out0, out1, idx_s, idx_d, ra, rb, acc,
             sema, semb, semz):
    c = lax.axis_index("c")
    s = lax.axis_index("s")
    w = c * NS + s
    soff = pl.multiple_of(w * NSTEP * CHUNK, 8)
    pltpu.async_copy(src_ref.at[pl.ds(soff, NSTEP * CHUNK)], idx_s, sema)
    pltpu.async_copy(dst_ref.at[w], idx_d, semb)

    def sidx(j):
        return idx_s.at[pl.ds(pl.multiple_of(j * CHUNK, 8), CHUNK)]

    zero16 = jnp.zeros((16,), F32)

    def zfill(r, carry):
        for l in range(D // 16):
            rb[r, pl.ds(l * 16, 16)] = zero16
        return carry

    lax.fori_loop(0, CHUNK, zfill, 0)

    for k in range(RPS // CHUNK):
        pltpu.async_copy(rb, acc.at[pl.ds(s * RPS + k * CHUNK, CHUNK)], semz)
    for k in range(RPS // CHUNK):
        pltpu.make_async_copy(rb, acc.at[pl.ds(s * RPS + k * CHUNK, CHUNK)],
                              semz).wait()
    pltpu.make_async_copy(src_ref.at[pl.ds(soff, NSTEP * CHUNK)], idx_s,
                          sema).wait()
    pltpu.make_async_copy(dst_ref.at[w], idx_d, semb).wait()
    plsc.subcore_barrier()

    pltpu.async_copy(g_ref.at[sidx(0)], ra, sema)

    def body(i, carry):
        j0 = 2 * i
        pltpu.async_copy(g_ref.at[sidx(j0 + 1)], rb, semb)
        pltpu.make_async_copy(g_ref.at[sidx(j0)], ra, sema).wait()
        pltpu.sync_copy(ra, acc.at[idx_d.at[j0]], add=True)
        pltpu.async_copy(g_ref.at[sidx(j0 + 2)], ra, sema)
        pltpu.make_async_copy(g_ref.at[sidx(j0 + 1)], rb, semb).wait()
        pltpu.sync_copy(rb, acc.at[idx_d.at[j0 + 1]], add=True)
        return carry

    lax.fori_loop(0, (NSTEP - 1) // 2, body, 0)
    pltpu.make_async_copy(g_ref.at[sidx(NSTEP - 1)], ra, sema).wait()
    pltpu.sync_copy(ra, acc.at[idx_d.at[NSTEP - 1]], add=True)
    plsc.subcore_barrier()

    def do_wb(out_ref):
        bufs = (ra, rb)
        nk = RPS // CHUNK
        for k in range(nk):
            buf = bufs[k % 2]
            off = s * RPS + k * CHUNK
            if k >= 2:
                poff = s * RPS + (k - 2) * CHUNK
                pltpu.make_async_copy(buf, out_ref.at[pl.ds(poff, CHUNK)],
                                      semz).wait()
            pltpu.sync_copy(acc.at[pl.ds(off, CHUNK)], buf)
            pltpu.async_copy(buf, out_ref.at[pl.ds(off, CHUNK)], semz)
        for k in (nk - 2, nk - 1):
            off = s * RPS + k * CHUNK
            pltpu.make_async_copy(bufs[k % 2], out_ref.at[pl.ds(off, CHUNK)],
                                  semz).wait()

    @pl.when(c == 0)
    def _():
        do_wb(out0)

    @pl.when(c == 1)
    def _():
        do_wb(out1)


BM = 2000
GRID = N // BM

_row = lambda i: (i, 0)
_all = lambda i: (0, 0)


def _tc_first_body(x_ref, w_ref, d0_ref, d1_ref, g_ref):
    dinv = lax.rsqrt(d0_ref[...] + d1_ref[...] + 1.0)
    h = jnp.dot(x_ref[...], w_ref[...], preferred_element_type=F32,
                precision=lax.Precision.HIGHEST)
    g_ref[...] = h * dinv


_tc_first = pl.pallas_call(
    _tc_first_body,
    grid=(GRID,),
    in_specs=[
        pl.BlockSpec((BM, D), _row),
        pl.BlockSpec((D, D), _all),
        pl.BlockSpec((BM, 1), _row),
        pl.BlockSpec((BM, 1), _row),
    ],
    out_specs=pl.BlockSpec((BM, D), _row),
    out_shape=jax.ShapeDtypeStruct((N, D), F32),
)


def _tc_mid_body(a0_ref, a1_ref, g1_ref, d0_ref, d1_ref, b_ref, w_ref, g2_ref):
    dinv = lax.rsqrt(d0_ref[...] + d1_ref[...] + 1.0)
    h = (a0_ref[...] + a1_ref[...] + g1_ref[...]) * dinv + b_ref[...]
    h = jnp.maximum(h, 0.0)
    h2 = jnp.dot(h, w_ref[...], preferred_element_type=F32,
                 precision=lax.Precision.HIGHEST)
    g2_ref[...] = h2 * dinv


_tc_mid = pl.pallas_call(
    _tc_mid_body,
    grid=(GRID,),
    in_specs=[
        pl.BlockSpec((BM, D), _row),
        pl.BlockSpec((BM, D), _row),
        pl.BlockSpec((BM, D), _row),
        pl.BlockSpec((BM, 1), _row),
        pl.BlockSpec((BM, 1), _row),
        pl.BlockSpec((1, D), _all),
        pl.BlockSpec((D, D), _all),
    ],
    out_specs=pl.BlockSpec((BM, D), _row),
    out_shape=jax.ShapeDtypeStruct((N, D), F32),
)


def _tc_final_body(c0_ref, c1_ref, g2_ref, d0_ref, d1_ref, b_ref, o_ref):
    dinv = lax.rsqrt(d0_ref[...] + d1_ref[...] + 1.0)
    z = (c0_ref[...] + c1_ref[...] + g2_ref[...]) * dinv + b_ref[...]
    z = z - jnp.max(z, axis=1, keepdims=True)
    o_ref[...] = z - jnp.log(jnp.sum(jnp.exp(z), axis=1, keepdims=True))


_tc_final = pl.pallas_call(
    _tc_final_body,
    grid=(GRID,),
    in_specs=[
        pl.BlockSpec((BM, D), _row),
        pl.BlockSpec((BM, D), _row),
        pl.BlockSpec((BM, D), _row),
        pl.BlockSpec((BM, 1), _row),
        pl.BlockSpec((BM, 1), _row),
        pl.BlockSpec((1, D), _all),
    ],
    out_specs=pl.BlockSpec((BM, D), _row),
    out_shape=jax.ShapeDtypeStruct((N, D), F32),
)


@jax.jit
def _run(x, edge_index, W1, b1, W2, b2):
    src = edge_index[0]
    dst = edge_index[1]
    src2 = src
    dst2 = dst.reshape(NW, NSTEP, CHUNK)

    d0, d1 = _sc_degree(dst2)
    d0 = d0.reshape(NPAD, 1)
    d1 = d1.reshape(NPAD, 1)

    g1 = _tc_first(x, W1, d0, d1)
    a0, a1 = _sc_prop(src2, dst2, g1)
    g2 = _tc_mid(a0, a1, g1, d0, d1, b1.reshape(1, D), W2)
    c0, c1 = _sc_prop(src2, dst2, g2)
    out = _tc_final(c0, c1, g2, d0, d1, b2.reshape(1, D))
    return out


def kernel(x, edge_index, W1, b1, W2, b2):
    return _run(x, edge_index, W1, b1, W2, b2)


# speedup vs baseline: 1.0908x; 1.0908x over previous
import functools

import jax
import jax.numpy as jnp
from jax import lax
from jax.experimental import pallas as pl
from jax.experimental.pallas import tpu as pltpu
from jax.experimental.pallas import tpu_sc as plsc

N = 10000
D = 128
E = 320000

NC = 2
NS = 16
NW = NC * NS
NPAD = 10240
CHUNK = 128
EPW = 10240
NSTEP = EPW // CHUNK
EPAD = EPW * NW
RPS = NPAD // NS
F32 = jnp.float32

_mesh = plsc.VectorSubcoreMesh(core_axis_name="c", subcore_axis_name="s")


@functools.partial(
    pl.kernel,
    out_type=(
        jax.ShapeDtypeStruct((NPAD,), F32),
        jax.ShapeDtypeStruct((NPAD,), F32),
    ),
    mesh=_mesh,
    scratch_types=[
        pltpu.VMEM((NSTEP, CHUNK), jnp.int32),
        pltpu.VMEM((CHUNK,), F32),
        pltpu.VMEM((RPS,), F32),
        pltpu.VMEM_SHARED((NPAD,), F32),
    ],
)
def _sc_degree(dst_ref, out0, out1, idx_d, ones, zbuf, degacc):
    c = lax.axis_index("c")
    s = lax.axis_index("s")
    w = c * NS + s
    pltpu.sync_copy(dst_ref.at[pl.ds(w * NSTEP, NSTEP)], idx_d)

    one16 = jnp.ones((16,), F32)
    zero16 = jnp.zeros((16,), F32)
    for l in range(CHUNK // 16):
        ones[pl.ds(l * 16, 16)] = one16

    def zfill(k, carry):
        zbuf[pl.ds(pl.multiple_of(k * 16, 16), 16)] = zero16
        return carry

    lax.fori_loop(0, RPS // 16, zfill, 0)
    pltpu.sync_copy(zbuf, degacc.at[pl.ds(s * RPS, RPS)])
    plsc.subcore_barrier()

    def body(i, carry):
        pltpu.sync_copy(ones, degacc.at[idx_d.at[i]], add=True)
        return carry

    lax.fori_loop(0, NSTEP, body, 0)
    plsc.subcore_barrier()

    pltpu.sync_copy(degacc.at[pl.ds(s * RPS, RPS)], zbuf)

    @pl.when(c == 0)
    def _():
        pltpu.sync_copy(zbuf, out0.at[pl.ds(s * RPS, RPS)])

    @pl.when(c == 1)
    def _():
        pltpu.sync_copy(zbuf, out1.at[pl.ds(s * RPS, RPS)])


@functools.partial(
    pl.kernel,
    out_type=(
        jax.ShapeDtypeStruct((NPAD, D), F32),
        jax.ShapeDtypeStruct((NPAD, D), F32),
    ),
    mesh=_mesh,
    scratch_types=[
        pltpu.VMEM((EPW,), jnp.int32),
        pltpu.VMEM((NSTEP // 2, CHUNK), jnp.int32),
        pltpu.VMEM((CHUNK, D), F32),
        pltpu.VMEM((CHUNK, D), F32),
        pltpu.VMEM_SHARED((NPAD, D), F32),
        pltpu.SemaphoreType.DMA,
        pltpu.SemaphoreType.DMA,
        pltpu.SemaphoreType.DMA,
    ],
)
def _sc_prop(src_ref, dst_ref, g_ref, out0, out1, idx_s, idx_d, ra, rb, acc,
             sema, semb, semz):
    HSTEP = NSTEP // 2
    c = lax.axis_index("c")
    s = lax.axis_index("s")
    w = c * NS + s
    soff = pl.multiple_of(w * EPW, 8)
    pltpu.async_copy(src_ref.at[pl.ds(soff, EPW)], idx_s, sema)
    pltpu.async_copy(dst_ref.at[pl.ds(w * NSTEP, HSTEP)], idx_d, semb)

    def sidx(j):
        return idx_s.at[pl.ds(pl.multiple_of(j * CHUNK, 8), CHUNK)]

    zero16 = jnp.zeros((16,), F32)

    def zfill(r, carry):
        for l in range(D // 16):
            rb[r, pl.ds(l * 16, 16)] = zero16
        return carry

    lax.fori_loop(0, CHUNK, zfill, 0)

    for k in range(RPS // CHUNK):
        pltpu.async_copy(rb, acc.at[pl.ds(s * RPS + k * CHUNK, CHUNK)], semz)
    for k in range(RPS // CHUNK):
        pltpu.make_async_copy(rb, acc.at[pl.ds(s * RPS + k * CHUNK, CHUNK)],
                              semz).wait()
    pltpu.make_async_copy(src_ref.at[pl.ds(soff, EPW)], idx_s, sema).wait()
    pltpu.make_async_copy(dst_ref.at[pl.ds(w * NSTEP, HSTEP)], idx_d,
                          semb).wait()
    plsc.subcore_barrier()

    pltpu.async_copy(g_ref.at[sidx(0)], ra, sema)

    def body(i, carry):
        j0 = 2 * i
        r0 = lax.rem(j0, HSTEP)

        @pl.when(j0 == HSTEP)
        def _():
            pltpu.sync_copy(dst_ref.at[pl.ds(w * NSTEP + HSTEP, HSTEP)],
                            idx_d)

        pltpu.async_copy(g_ref.at[sidx(j0 + 1)], rb, semb)
        pltpu.make_async_copy(g_ref.at[sidx(j0)], ra, sema).wait()
        pltpu.sync_copy(ra, acc.at[idx_d.at[r0]], add=True)

        @pl.when(i < NSTEP // 2 - 1)
        def _():
            pltpu.async_copy(g_ref.at[sidx(j0 + 2)], ra, sema)

        pltpu.make_async_copy(g_ref.at[sidx(j0 + 1)], rb, semb).wait()
        pltpu.sync_copy(rb, acc.at[idx_d.at[r0 + 1]], add=True)
        return carry

    lax.fori_loop(0, NSTEP // 2, body, 0)
    plsc.subcore_barrier()

    def do_wb(out_ref):
        bufs = (ra, rb)
        nk = RPS // CHUNK
        for k in range(nk):
            buf = bufs[k % 2]
            off = s * RPS + k * CHUNK
            if k >= 2:
                poff = s * RPS + (k - 2) * CHUNK
                pltpu.make_async_copy(buf, out_ref.at[pl.ds(poff, CHUNK)],
                                      semz).wait()
            pltpu.sync_copy(acc.at[pl.ds(off, CHUNK)], buf)
            pltpu.async_copy(buf, out_ref.at[pl.ds(off, CHUNK)], semz)
        for k in range(nk - 2, nk):
            buf = bufs[k % 2]
            off = s * RPS + k * CHUNK
            pltpu.make_async_copy(buf, out_ref.at[pl.ds(off, CHUNK)],
                                  semz).wait()

    @pl.when(c == 0)
    def _():
        do_wb(out0)

    @pl.when(c == 1)
    def _():
        do_wb(out1)


BM = 2000
GRID = N // BM

_row = lambda i: (i, 0)
_all = lambda i: (0, 0)


def _tc_first_body(x_ref, w_ref, d0_ref, d1_ref, g_ref):
    dinv = lax.rsqrt(d0_ref[...] + d1_ref[...] + 1.0)
    h = jnp.dot(x_ref[...], w_ref[...], preferred_element_type=F32,
                precision=lax.Precision.HIGHEST)
    g_ref[...] = h * dinv


_tc_first = pl.pallas_call(
    _tc_first_body,
    grid=(GRID,),
    in_specs=[
        pl.BlockSpec((BM, D), _row),
        pl.BlockSpec((D, D), _all),
        pl.BlockSpec((BM, 1), _row),
        pl.BlockSpec((BM, 1), _row),
    ],
    out_specs=pl.BlockSpec((BM, D), _row),
    out_shape=jax.ShapeDtypeStruct((N, D), F32),
)


def _tc_mid_body(a0_ref, a1_ref, g1_ref, d0_ref, d1_ref, b_ref, w_ref, g2_ref):
    dinv = lax.rsqrt(d0_ref[...] + d1_ref[...] + 1.0)
    h = (a0_ref[...] + a1_ref[...] + g1_ref[...]) * dinv + b_ref[...]
    h = jnp.maximum(h, 0.0)
    h2 = jnp.dot(h, w_ref[...], preferred_element_type=F32,
                 precision=lax.Precision.HIGHEST)
    g2_ref[...] = h2 * dinv


_tc_mid = pl.pallas_call(
    _tc_mid_body,
    grid=(GRID,),
    in_specs=[
        pl.BlockSpec((BM, D), _row),
        pl.BlockSpec((BM, D), _row),
        pl.BlockSpec((BM, D), _row),
        pl.BlockSpec((BM, 1), _row),
        pl.BlockSpec((BM, 1), _row),
        pl.BlockSpec((1, D), _all),
        pl.BlockSpec((D, D), _all),
    ],
    out_specs=pl.BlockSpec((BM, D), _row),
    out_shape=jax.ShapeDtypeStruct((N, D), F32),
)


def _tc_final_body(c0_ref, c1_ref, g2_ref, d0_ref, d1_ref, b_ref, o_ref):
    dinv = lax.rsqrt(d0_ref[...] + d1_ref[...] + 1.0)
    z = (c0_ref[...] + c1_ref[...] + g2_ref[...]) * dinv + b_ref[...]
    z = z - jnp.max(z, axis=1, keepdims=True)
    o_ref[...] = z - jnp.log(jnp.sum(jnp.exp(z), axis=1, keepdims=True))


_tc_final = pl.pallas_call(
    _tc_final_body,
    grid=(GRID,),
    in_specs=[
        pl.BlockSpec((BM, D), _row),
        pl.BlockSpec((BM, D), _row),
        pl.BlockSpec((BM, D), _row),
        pl.BlockSpec((BM, 1), _row),
        pl.BlockSpec((BM, 1), _row),
        pl.BlockSpec((1, D), _all),
    ],
    out_specs=pl.BlockSpec((BM, D), _row),
    out_shape=jax.ShapeDtypeStruct((N, D), F32),
)


@jax.jit
def _run(x, edge_index, W1, b1, W2, b2):
    src = edge_index[0]
    dst = edge_index[1]
    fill_d = N + jnp.arange(EPAD - E, dtype=jnp.int32) % (NPAD - N)
    fill_s = jnp.arange(EPAD - E, dtype=jnp.int32) % N
    src2 = jnp.concatenate([src, fill_s])
    dst2 = jnp.concatenate([dst, fill_d]).reshape(EPAD // CHUNK, CHUNK)

    d0, d1 = _sc_degree(dst2)
    d0 = d0.reshape(NPAD, 1)
    d1 = d1.reshape(NPAD, 1)

    g1 = _tc_first(x, W1, d0, d1)
    a0, a1 = _sc_prop(src2, dst2, g1)
    g2 = _tc_mid(a0, a1, g1, d0, d1, b1.reshape(1, D), W2)
    c0, c1 = _sc_prop(src2, dst2, g2)
    out = _tc_final(c0, c1, g2, d0, d1, b2.reshape(1, D))
    return out


def kernel(x, edge_index, W1, b1, W2, b2):
    return _run(x, edge_index, W1, b1, W2, b2)


# speedup vs baseline: 1.0941x; 1.0030x over previous
import functools

import jax
import jax.numpy as jnp
from jax import lax
from jax.experimental import pallas as pl
from jax.experimental.pallas import tpu as pltpu
from jax.experimental.pallas import tpu_sc as plsc

N = 10000
D = 128
E = 320000

NC = 2
NS = 16
NW = NC * NS
NPAD = 10240
CHUNK = 128
EPW = 10240
NSTEP = EPW // CHUNK
EPAD = EPW * NW
RPS = NPAD // NS
F32 = jnp.float32

_mesh = plsc.VectorSubcoreMesh(core_axis_name="c", subcore_axis_name="s")


@functools.partial(
    pl.kernel,
    out_type=(
        jax.ShapeDtypeStruct((NPAD,), F32),
        jax.ShapeDtypeStruct((NPAD,), F32),
    ),
    mesh=_mesh,
    scratch_types=[
        pltpu.VMEM((NSTEP, CHUNK), jnp.int32),
        pltpu.VMEM((CHUNK,), F32),
        pltpu.VMEM((RPS,), F32),
        pltpu.VMEM_SHARED((NPAD,), F32),
    ],
)
def _sc_degree(dst_ref, out0, out1, idx_d, ones, zbuf, degacc):
    c = lax.axis_index("c")
    s = lax.axis_index("s")
    w = c * NS + s
    pltpu.sync_copy(dst_ref.at[pl.ds(w * NSTEP, NSTEP)], idx_d)

    one16 = jnp.ones((16,), F32)
    zero16 = jnp.zeros((16,), F32)
    for l in range(CHUNK // 16):
        ones[pl.ds(l * 16, 16)] = one16

    def zfill(k, carry):
        zbuf[pl.ds(pl.multiple_of(k * 16, 16), 16)] = zero16
        return carry

    lax.fori_loop(0, RPS // 16, zfill, 0)
    pltpu.sync_copy(zbuf, degacc.at[pl.ds(s * RPS, RPS)])
    plsc.subcore_barrier()

    def body(i, carry):
        pltpu.sync_copy(ones, degacc.at[idx_d.at[i]], add=True)
        return carry

    lax.fori_loop(0, NSTEP, body, 0)
    plsc.subcore_barrier()

    pltpu.sync_copy(degacc.at[pl.ds(s * RPS, RPS)], zbuf)

    @pl.when(c == 0)
    def _():
        pltpu.sync_copy(zbuf, out0.at[pl.ds(s * RPS, RPS)])

    @pl.when(c == 1)
    def _():
        pltpu.sync_copy(zbuf, out1.at[pl.ds(s * RPS, RPS)])


@functools.partial(
    pl.kernel,
    out_type=(
        jax.ShapeDtypeStruct((NPAD, D), F32),
        jax.ShapeDtypeStruct((NPAD, D), F32),
    ),
    mesh=_mesh,
    scratch_types=[
        pltpu.VMEM((EPW,), jnp.int32),
        pltpu.VMEM((NSTEP // 2, CHUNK), jnp.int32),
        pltpu.VMEM((CHUNK, D), F32),
        pltpu.VMEM((CHUNK, D), F32),
        pltpu.VMEM_SHARED((NPAD, D), F32),
        pltpu.SemaphoreType.DMA,
        pltpu.SemaphoreType.DMA,
        pltpu.SemaphoreType.DMA,
    ],
)
def _sc_prop(src_ref, dst_ref, g_ref, out0, out1, idx_s, idx_d, ra, rb, acc,
             sema, semb, semz):
    HSTEP = NSTEP // 2
    c = lax.axis_index("c")
    s = lax.axis_index("s")
    w = c * NS + s
    soff = pl.multiple_of(w * EPW, 8)
    pltpu.async_copy(src_ref.at[pl.ds(soff, EPW)], idx_s, sema)
    pltpu.async_copy(dst_ref.at[pl.ds(w * NSTEP, HSTEP)], idx_d, semb)

    def sidx(j):
        return idx_s.at[pl.ds(pl.multiple_of(j * CHUNK, 8), CHUNK)]

    zero16 = jnp.zeros((16,), F32)

    def zfill(r, carry):
        for l in range(D // 16):
            rb[r, pl.ds(l * 16, 16)] = zero16
        return carry

    lax.fori_loop(0, CHUNK, zfill, 0)

    for k in range(RPS // CHUNK):
        pltpu.async_copy(rb, acc.at[pl.ds(s * RPS + k * CHUNK, CHUNK)], semz)
    for k in range(RPS // CHUNK):
        pltpu.make_async_copy(rb, acc.at[pl.ds(s * RPS + k * CHUNK, CHUNK)],
                              semz).wait()
    pltpu.make_async_copy(src_ref.at[pl.ds(soff, EPW)], idx_s, sema).wait()
    pltpu.make_async_copy(dst_ref.at[pl.ds(w * NSTEP, HSTEP)], idx_d,
                          semb).wait()
    plsc.subcore_barrier()

    pltpu.async_copy(g_ref.at[sidx(0)], ra, sema)

    def body(i, carry):
        j0 = 2 * i
        r0 = lax.rem(j0, HSTEP)

        @pl.when(j0 == HSTEP)
        def _():
            pltpu.sync_copy(dst_ref.at[pl.ds(w * NSTEP + HSTEP, HSTEP)],
                            idx_d)

        pltpu.async_copy(g_ref.at[sidx(j0 + 1)], rb, semb)
        pltpu.make_async_copy(g_ref.at[sidx(j0)], ra, sema).wait()
        pltpu.sync_copy(ra, acc.at[idx_d.at[r0]], add=True)

        @pl.when(i < NSTEP // 2 - 1)
        def _():
            pltpu.async_copy(g_ref.at[sidx(j0 + 2)], ra, sema)

        pltpu.make_async_copy(g_ref.at[sidx(j0 + 1)], rb, semb).wait()
        pltpu.sync_copy(rb, acc.at[idx_d.at[r0 + 1]], add=True)
        return carry

    lax.fori_loop(0, NSTEP // 2, body, 0)
    plsc.subcore_barrier()

    def do_wb(out_ref):
        bufs = (ra, rb)
        nk = RPS // CHUNK
        for k in range(nk):
            buf = bufs[k % 2]
            off = s * RPS + k * CHUNK
            if k >= 2:
                poff = s * RPS + (k - 2) * CHUNK
                pltpu.make_async_copy(buf, out_ref.at[pl.ds(poff, CHUNK)],
                                      semz).wait()
            pltpu.sync_copy(acc.at[pl.ds(off, CHUNK)], buf)
            pltpu.async_copy(buf, out_ref.at[pl.ds(off, CHUNK)], semz)
        for k in range(nk - 2, nk):
            buf = bufs[k % 2]
            off = s * RPS + k * CHUNK
            pltpu.make_async_copy(buf, out_ref.at[pl.ds(off, CHUNK)],
                                  semz).wait()

    @pl.when(c == 0)
    def _():
        do_wb(out0)

    @pl.when(c == 1)
    def _():
        do_wb(out1)


BM = 2000
GRID = N // BM

_row = lambda i: (i, 0)
_all = lambda i: (0, 0)


def _tc_mm_body(x_ref, w_ref, h_ref):
    h_ref[...] = jnp.dot(x_ref[...], w_ref[...], preferred_element_type=F32,
                         precision=lax.Precision.HIGHEST)


_tc_mm = pl.pallas_call(
    _tc_mm_body,
    grid=(GRID,),
    in_specs=[
        pl.BlockSpec((BM, D), _row),
        pl.BlockSpec((D, D), _all),
    ],
    out_specs=pl.BlockSpec((BM, D), _row),
    out_shape=jax.ShapeDtypeStruct((N, D), F32),
)


def _tc_scale_body(h_ref, d0_ref, d1_ref, g_ref):
    dinv = lax.rsqrt(d0_ref[...] + d1_ref[...] + 1.0)
    g_ref[...] = h_ref[...] * dinv


_tc_scale = pl.pallas_call(
    _tc_scale_body,
    grid=(GRID,),
    in_specs=[
        pl.BlockSpec((BM, D), _row),
        pl.BlockSpec((BM, 1), _row),
        pl.BlockSpec((BM, 1), _row),
    ],
    out_specs=pl.BlockSpec((BM, D), _row),
    out_shape=jax.ShapeDtypeStruct((N, D), F32),
)


def _tc_mid_body(a0_ref, a1_ref, g1_ref, d0_ref, d1_ref, b_ref, w_ref, g2_ref):
    dinv = lax.rsqrt(d0_ref[...] + d1_ref[...] + 1.0)
    h = (a0_ref[...] + a1_ref[...] + g1_ref[...]) * dinv + b_ref[...]
    h = jnp.maximum(h, 0.0)
    h2 = jnp.dot(h, w_ref[...], preferred_element_type=F32,
                 precision=lax.Precision.HIGHEST)
    g2_ref[...] = h2 * dinv


_tc_mid = pl.pallas_call(
    _tc_mid_body,
    grid=(GRID,),
    in_specs=[
        pl.BlockSpec((BM, D), _row),
        pl.BlockSpec((BM, D), _row),
        pl.BlockSpec((BM, D), _row),
        pl.BlockSpec((BM, 1), _row),
        pl.BlockSpec((BM, 1), _row),
        pl.BlockSpec((1, D), _all),
        pl.BlockSpec((D, D), _all),
    ],
    out_specs=pl.BlockSpec((BM, D), _row),
    out_shape=jax.ShapeDtypeStruct((N, D), F32),
)


def _tc_final_body(c0_ref, c1_ref, g2_ref, d0_ref, d1_ref, b_ref, o_ref):
    dinv = lax.rsqrt(d0_ref[...] + d1_ref[...] + 1.0)
    z = (c0_ref[...] + c1_ref[...] + g2_ref[...]) * dinv + b_ref[...]
    z = z - jnp.max(z, axis=1, keepdims=True)
    o_ref[...] = z - jnp.log(jnp.sum(jnp.exp(z), axis=1, keepdims=True))


_tc_final = pl.pallas_call(
    _tc_final_body,
    grid=(GRID,),
    in_specs=[
        pl.BlockSpec((BM, D), _row),
        pl.BlockSpec((BM, D), _row),
        pl.BlockSpec((BM, D), _row),
        pl.BlockSpec((BM, 1), _row),
        pl.BlockSpec((BM, 1), _row),
        pl.BlockSpec((1, D), _all),
    ],
    out_specs=pl.BlockSpec((BM, D), _row),
    out_shape=jax.ShapeDtypeStruct((N, D), F32),
)


@jax.jit
def _run(x, edge_index, W1, b1, W2, b2):
    src = edge_index[0]
    dst = edge_index[1]
    fill_d = N + jnp.arange(EPAD - E, dtype=jnp.int32) % (NPAD - N)
    fill_s = jnp.arange(EPAD - E, dtype=jnp.int32) % N
    src2 = jnp.concatenate([src, fill_s])
    dst2 = jnp.concatenate([dst, fill_d]).reshape(EPAD // CHUNK, CHUNK)

    h1 = _tc_mm(x, W1)
    d0, d1 = _sc_degree(dst2)
    d0 = d0.reshape(NPAD, 1)
    d1 = d1.reshape(NPAD, 1)

    g1 = _tc_scale(h1, d0, d1)
    a0, a1 = _sc_prop(src2, dst2, g1)
    g2 = _tc_mid(a0, a1, g1, d0, d1, b1.reshape(1, D), W2)
    c0, c1 = _sc_prop(src2, dst2, g2)
    out = _tc_final(c0, c1, g2, d0, d1, b2.reshape(1, D))
    return out


def kernel(x, edge_index, W1, b1, W2, b2):
    return _run(x, edge_index, W1, b1, W2, b2)
